# baseline (speedup 1.0000x reference)
import functools

import jax
import jax.numpy as jnp
from jax import lax
from jax.experimental import pallas as pl
from jax.experimental.pallas import tpu as pltpu
from jax.experimental.pallas import tpu_sc as plsc

_NUM_CORES = 2
_NUM_SUBCORES = 16
_NUM_WORKERS = _NUM_CORES * _NUM_SUBCORES

_NBUF = 4
_OUT_MINOR = 128


def kernel(input_ids, table):
    batch, hist = input_ids.shape
    vocab, dim = table.shape
    assert batch % _NUM_WORKERS == 0
    rows_per_w = batch // _NUM_WORKERS

    idx_flat = input_ids.reshape(batch * hist)

    mesh = plsc.VectorSubcoreMesh(
        core_axis_name="c", subcore_axis_name="s",
        num_cores=_NUM_CORES, num_subcores=_NUM_SUBCORES)

    @functools.partial(
        pl.kernel,
        out_type=jax.ShapeDtypeStruct((batch, hist, _OUT_MINOR), jnp.float32),
        mesh=mesh,
        scratch_types=[
            pltpu.VMEM((rows_per_w * hist,), jnp.int32),
            [pltpu.VMEM((hist, dim), jnp.float32) for _ in range(_NBUF)],
            [pltpu.SemaphoreType.DMA for _ in range(_NBUF)],
            [pltpu.SemaphoreType.DMA for _ in range(_NBUF)],
        ],
        compiler_params=pltpu.CompilerParams(
            use_tc_tiling_on_sc=False,
            allow_input_fusion=(True, True),
        ),
    )
    def gather_kernel(table_hbm, idx_hbm, out_hbm, idx_v, bufs, gsems, osems):
        wid = lax.axis_index("s") * _NUM_CORES + lax.axis_index("c")
        base = wid * rows_per_w
        pltpu.sync_copy(
            idx_hbm.at[pl.ds(base * hist, rows_per_w * hist)], idx_v)

        def gather_desc(r, b):
            return pltpu.make_async_copy(
                table_hbm.at[idx_v.at[pl.ds(r * hist, hist)]],
                bufs[b], gsems[b])

        def out_desc(r, b):
            return pltpu.make_async_copy(
                bufs[b], out_hbm.at[base + r, :, pl.ds(0, dim)], osems[b])

        for b in range(_NBUF):
            gather_desc(b, b).start()

        def round_body(i, carry):
            r0 = i * _NBUF
            for b in range(_NBUF):
                gather_desc(r0 + b, b).wait()
                out_desc(r0 + b, b).start()
            for b in range(_NBUF):
                out_desc(r0 + b, b).wait()
                gather_desc(r0 + b + _NBUF, b).start()
            return carry

        lax.fori_loop(0, (rows_per_w - _NBUF) // _NBUF, round_body, 0,
                      unroll=False)

        r0 = rows_per_w - _NBUF
        for b in range(_NBUF):
            gather_desc(r0 + b, b).wait()
            out_desc(r0 + b, b).start()
        for b in range(_NBUF):
            out_desc(r0 + b, b).wait()

    out = gather_kernel(table, idx_flat)
    return out[:, :, :dim]


# speedup vs baseline: 1.0016x; 1.0016x over previous
import functools

import jax
import jax.numpy as jnp
from jax import lax
from jax.experimental import pallas as pl
from jax.experimental.pallas import tpu as pltpu
from jax.experimental.pallas import tpu_sc as plsc

_NUM_CORES = 2
_NUM_SUBCORES = 16
_NUM_WORKERS = _NUM_CORES * _NUM_SUBCORES

_NBUF = 4
_OUT_MINOR = 128


def kernel(input_ids, table):
    batch, hist = input_ids.shape
    vocab, dim = table.shape
    assert batch % _NUM_WORKERS == 0
    rows_per_w = batch // _NUM_WORKERS

    idx_flat = input_ids.reshape(batch * hist)

    mesh = plsc.VectorSubcoreMesh(
        core_axis_name="c", subcore_axis_name="s",
        num_cores=_NUM_CORES, num_subcores=_NUM_SUBCORES)

    @functools.partial(
        pl.kernel,
        out_type=jax.ShapeDtypeStruct((batch, hist, _OUT_MINOR), jnp.float32),
        mesh=mesh,
        scratch_types=[
            pltpu.VMEM((rows_per_w * hist,), jnp.int32),
            [pltpu.VMEM((hist, dim), jnp.float32) for _ in range(_NBUF)],
            [pltpu.SemaphoreType.DMA for _ in range(_NBUF)],
            [pltpu.SemaphoreType.DMA for _ in range(_NBUF)],
        ],
        compiler_params=pltpu.CompilerParams(use_tc_tiling_on_sc=False),
    )
    def gather_kernel(table_hbm, idx_hbm, out_hbm, idx_v, bufs, gsems, osems):
        wid = lax.axis_index("s") * _NUM_CORES + lax.axis_index("c")
        base = wid * rows_per_w
        pltpu.sync_copy(
            idx_hbm.at[pl.ds(base * hist, rows_per_w * hist)], idx_v)

        def gather_desc(r, b):
            return pltpu.make_async_copy(
                table_hbm.at[idx_v.at[pl.ds(r * hist, hist)]],
                bufs[b], gsems[b])

        def out_desc(r, b):
            return pltpu.make_async_copy(
                bufs[b], out_hbm.at[base + r, :, pl.ds(0, dim)], osems[b])

        for b in range(_NBUF):
            gather_desc(b, b).start()

        def round_body(i, carry):
            r0 = i * _NBUF
            for b in range(_NBUF):
                gather_desc(r0 + b, b).wait()
                out_desc(r0 + b, b).start()
            for b in range(_NBUF):
                out_desc(r0 + b, b).wait()
                gather_desc(r0 + b + _NBUF, b).start()
            return carry

        lax.fori_loop(0, (rows_per_w - _NBUF) // _NBUF, round_body, 0,
                      unroll=False)

        r0 = rows_per_w - _NBUF
        for b in range(_NBUF):
            gather_desc(r0 + b, b).wait()
            out_desc(r0 + b, b).start()
        for b in range(_NBUF):
            out_desc(r0 + b, b).wait()

    out = gather_kernel(table, idx_flat)
    return out[:, :, :dim]
